# Initial kernel scaffold; baseline (speedup 1.0000x reference)
import jax
import jax.numpy as jnp
from jax.experimental import pallas as pl


def kernel(x, V, E, degV, degE, W0, b0, Wc, Wout, bout):
    raise NotImplementedError("write your pallas kernel here")


# speedup vs baseline: 3.4308x; 3.4308x over previous
import functools
import math

import jax
import jax.numpy as jnp
from jax import lax
from jax.experimental import pallas as pl
from jax.experimental.pallas import tpu as pltpu
from jax.experimental.pallas import tpu_sc as plsc

N = 10000
M = 5000
NNZ = 320000
F = 128
NCLASS = 40
NLAYER = 2

NC = 2
NS = 16
NW = NC * NS
BATCH = 128
NB = -(-NNZ // (NW * BATCH))
NNZ_PAD = NW * NB * BATCH
PAD = NNZ_PAD - NNZ

N_PAD = 10240
M_PAD = 5120
ME16 = M_PAD // NS
NV16 = N_PAD // NS

_mesh = plsc.VectorSubcoreMesh(core_axis_name="c", subcore_axis_name="s")


def _fill(ref, width, value):
    def row(r, carry):
        for k in range(width // 16):
            ref[r, pl.ds(k * 16, 16)] = jnp.full((16,), value, jnp.float32)
        return carry
    lax.fori_loop(0, BATCH, row, 0)


def _chunks(total):
    out, off = [], 0
    while off < total:
        sz = min(BATCH, total - off)
        out.append((off, sz))
        off += sz
    return out


@functools.partial(
    pl.kernel,
    out_type=jax.ShapeDtypeStruct((NC, M_PAD, F), jnp.float32),
    mesh=_mesh,
    scratch_types=[
        pltpu.VMEM((NB, BATCH), jnp.int32),
        pltpu.VMEM((NB, BATCH), jnp.int32),
        pltpu.VMEM((BATCH, F), jnp.float32),
        pltpu.SemaphoreType.DMA,
        pltpu.VMEM_SHARED((M_PAD, F), jnp.float32),
    ],
)
def _sc_phase1(h_hbm, vix_hbm, eix_hbm,
               xep_out, vix, eix, rows, sem, xe_sh):
    c = lax.axis_index("c")
    s = lax.axis_index("s")
    w = c * NS + s
    _fill(rows, F, 0.0)
    for off, sz in _chunks(ME16):
        pltpu.sync_copy(rows.at[pl.ds(0, sz), :],
                        xe_sh.at[pl.ds(s * ME16 + off, sz), :])
    pltpu.sync_copy(vix_hbm.at[w], vix)
    pltpu.sync_copy(eix_hbm.at[w], eix)
    plsc.subcore_barrier()

    def body(b, carry):
        pltpu.async_copy(h_hbm.at[vix.at[b]], rows, sem).wait()
        pltpu.sync_copy(rows, xe_sh.at[eix.at[b]], add=True)
        return carry

    lax.fori_loop(0, NB, body, 0)
    plsc.subcore_barrier()
    for off, sz in _chunks(ME16):
        pltpu.sync_copy(xe_sh.at[pl.ds(s * ME16 + off, sz), :],
                        rows.at[pl.ds(0, sz), :])
        pltpu.sync_copy(rows.at[pl.ds(0, sz), :],
                        xep_out.at[c, pl.ds(s * ME16 + off, sz), :])


@functools.partial(
    pl.kernel,
    out_type=jax.ShapeDtypeStruct((NC, M_PAD, F), jnp.float32),
    mesh=_mesh,
    scratch_types=[
        pltpu.VMEM((NB, BATCH), jnp.int32),
        pltpu.VMEM((BATCH, F), jnp.float32),
        pltpu.VMEM_SHARED((M_PAD, F), jnp.float32),
    ],
)
def _sc_cnt(eix_hbm, cnt_out, eix, ones_v, cnt_sh):
    c = lax.axis_index("c")
    s = lax.axis_index("s")
    w = c * NS + s
    _fill(ones_v, F, 0.0)
    for off, sz in _chunks(ME16):
        pltpu.sync_copy(ones_v.at[pl.ds(0, sz), :],
                        cnt_sh.at[pl.ds(s * ME16 + off, sz), :])
    _fill(ones_v, F, 1.0)
    pltpu.sync_copy(eix_hbm.at[w], eix)
    plsc.subcore_barrier()

    def body(b, carry):
        pltpu.sync_copy(ones_v, cnt_sh.at[eix.at[b]], add=True)
        return carry

    lax.fori_loop(0, NB, body, 0)
    plsc.subcore_barrier()
    for off, sz in _chunks(ME16):
        pltpu.sync_copy(cnt_sh.at[pl.ds(s * ME16 + off, sz), :],
                        ones_v.at[pl.ds(0, sz), :])
        pltpu.sync_copy(ones_v.at[pl.ds(0, sz), :],
                        cnt_out.at[c, pl.ds(s * ME16 + off, sz), :])


@functools.partial(
    pl.kernel,
    out_type=jax.ShapeDtypeStruct((NC, N_PAD, F), jnp.float32),
    mesh=_mesh,
    scratch_types=[
        pltpu.VMEM((NB, BATCH), jnp.int32),
        pltpu.VMEM((NB, BATCH), jnp.int32),
        pltpu.VMEM((BATCH, F), jnp.float32),
        pltpu.SemaphoreType.DMA,
        pltpu.VMEM_SHARED((N_PAD, F), jnp.float32),
    ],
)
def _sc_phase2(xe_hbm, vix_hbm, eix_hbm,
               xvp_out, vix, eix, rows, sem, xv_sh):
    c = lax.axis_index("c")
    s = lax.axis_index("s")
    w = c * NS + s
    _fill(rows, F, 0.0)
    for off, sz in _chunks(NV16):
        pltpu.sync_copy(rows.at[pl.ds(0, sz), :],
                        xv_sh.at[pl.ds(s * NV16 + off, sz), :])
    pltpu.sync_copy(vix_hbm.at[w], vix)
    pltpu.sync_copy(eix_hbm.at[w], eix)
    plsc.subcore_barrier()

    def body(b, carry):
        pltpu.async_copy(xe_hbm.at[eix.at[b]], rows, sem).wait()
        pltpu.sync_copy(rows, xv_sh.at[vix.at[b]], add=True)
        return carry

    lax.fori_loop(0, NB, body, 0)
    plsc.subcore_barrier()
    for off, sz in _chunks(NV16):
        pltpu.sync_copy(xv_sh.at[pl.ds(s * NV16 + off, sz), :],
                        rows)
        pltpu.sync_copy(rows,
                        xvp_out.at[c, pl.ds(s * NV16 + off, sz), :])


@functools.partial(
    pl.kernel,
    out_type=jax.ShapeDtypeStruct((N_PAD, F), jnp.float32),
    mesh=_mesh,
    scratch_types=[
        pltpu.VMEM((BATCH, F), jnp.float32),
        pltpu.VMEM_SHARED((N_PAD // NC, F), jnp.float32),
    ],
)
def _sc_roundtrip(h_hbm, out_hbm, buf, sh):
    c = lax.axis_index("c")
    s = lax.axis_index("s")
    w = c * NS + s
    npw = N_PAD // NW
    for off, sz in _chunks(npw):
        pltpu.sync_copy(h_hbm.at[pl.ds(w * npw + off, sz), :],
                        buf.at[pl.ds(0, sz), :])
        pltpu.sync_copy(buf.at[pl.ds(0, sz), :],
                        sh.at[pl.ds(s * npw + off, sz), :])
    plsc.subcore_barrier()
    for off, sz in _chunks(npw):
        pltpu.sync_copy(sh.at[pl.ds(s * npw + off, sz), :],
                        buf.at[pl.ds(0, sz), :])
        pltpu.sync_copy(buf.at[pl.ds(0, sz), :],
                        out_hbm.at[pl.ds(w * npw + off, sz), :])


@functools.partial(
    pl.kernel,
    out_type=jax.ShapeDtypeStruct((NNZ_PAD, F), jnp.float32),
    mesh=_mesh,
    scratch_types=[
        pltpu.VMEM((NB, BATCH), jnp.int32),
        pltpu.VMEM((BATCH, F), jnp.float32),
        pltpu.SemaphoreType.DMA,
    ],
)
def _sc_gather(h_hbm, vix_hbm, out_hbm, vix, rows, sem):
    c = lax.axis_index("c")
    s = lax.axis_index("s")
    w = c * NS + s
    pltpu.sync_copy(vix_hbm.at[w], vix)

    def body(b, carry):
        pltpu.async_copy(h_hbm.at[vix.at[b]], rows, sem).wait()
        pltpu.sync_copy(rows, out_hbm.at[pl.ds((w * NB + b) * BATCH, BATCH), :])
        return carry

    lax.fori_loop(0, NB, body, 0)


RB = 512


def _init_body(x_ref, w_ref, b_ref, o_ref):
    o_ref[...] = jax.nn.relu(
        jnp.dot(x_ref[...], w_ref[...], preferred_element_type=jnp.float32)
        + b_ref[...])


_tc_init = pl.pallas_call(
    _init_body,
    grid=(N_PAD // RB,),
    in_specs=[pl.BlockSpec((RB, F), lambda i: (i, 0)),
              pl.BlockSpec((F, F), lambda i: (0, 0)),
              pl.BlockSpec((1, F), lambda i: (0, 0))],
    out_specs=pl.BlockSpec((RB, F), lambda i: (i, 0)),
    out_shape=jax.ShapeDtypeStruct((N_PAD, F), jnp.float32),
)


def _combine_body(p_ref, c_ref, de_ref, o_ref):
    p = p_ref[0] + p_ref[1]
    cnt = c_ref[0, :, 0:1] + c_ref[1, :, 0:1]
    o_ref[...] = p * (de_ref[...] / jnp.maximum(cnt, 1.0))


_tc_combine = pl.pallas_call(
    _combine_body,
    grid=(M_PAD // RB,),
    in_specs=[pl.BlockSpec((NC, RB, F), lambda i: (0, i, 0)),
              pl.BlockSpec((NC, RB, F), lambda i: (0, i, 0)),
              pl.BlockSpec((RB, 1), lambda i: (i, 0))],
    out_specs=pl.BlockSpec((RB, F), lambda i: (i, 0)),
    out_shape=jax.ShapeDtypeStruct((M_PAD, F), jnp.float32),
)


def _update_body(p_ref, x0_ref, dv_ref, w_ref, o_ref, *, beta):
    xv = (p_ref[0] + p_ref[1]) * dv_ref[...]
    xi = 0.9 * xv + 0.1 * x0_ref[...]
    o_ref[...] = jax.nn.relu(
        (1.0 - beta) * xi
        + beta * jnp.dot(xi, w_ref[...], preferred_element_type=jnp.float32))


def _tc_update(beta):
    return pl.pallas_call(
        functools.partial(_update_body, beta=beta),
        grid=(N_PAD // RB,),
        in_specs=[pl.BlockSpec((NC, RB, F), lambda i: (0, i, 0)),
                  pl.BlockSpec((RB, F), lambda i: (i, 0)),
                  pl.BlockSpec((RB, 1), lambda i: (i, 0)),
                  pl.BlockSpec((F, F), lambda i: (0, 0))],
        out_specs=pl.BlockSpec((RB, F), lambda i: (i, 0)),
        out_shape=jax.ShapeDtypeStruct((N_PAD, F), jnp.float32),
    )


RO = 400


def _final_body(h_ref, w_ref, b_ref, o_ref):
    z = (jnp.dot(h_ref[...], w_ref[...], preferred_element_type=jnp.float32)
         + b_ref[...])
    m = jnp.max(z, axis=1, keepdims=True)
    lse = jnp.log(jnp.sum(jnp.exp(z - m), axis=1, keepdims=True))
    o_ref[...] = z - m - lse


_tc_final = pl.pallas_call(
    _final_body,
    grid=(N // RO,),
    in_specs=[pl.BlockSpec((RO, F), lambda i: (i, 0)),
              pl.BlockSpec((F, NCLASS), lambda i: (0, 0)),
              pl.BlockSpec((1, NCLASS), lambda i: (0, 0))],
    out_specs=pl.BlockSpec((RO, NCLASS), lambda i: (i, 0)),
    out_shape=jax.ShapeDtypeStruct((N, NCLASS), jnp.float32),
)


def kernel(x, V, E, degV, degE, W0, b0, Wc, Wout, bout):
    V32 = V.astype(jnp.int32)
    E32 = E.astype(jnp.int32)
    vp = jnp.concatenate([V32, jnp.full((PAD,), N, jnp.int32)]).reshape(NW, NB, BATCH)
    ep = jnp.concatenate([E32, jnp.full((PAD,), M, jnp.int32)]).reshape(NW, NB, BATCH)
    xp = jnp.pad(x, ((0, N_PAD - N), (0, 0)))
    degVp = jnp.pad(degV.astype(jnp.float32), ((0, N_PAD - N), (0, 0)))
    degEp = jnp.pad(degE.astype(jnp.float32), ((0, M_PAD - M), (0, 0)))
    h = _tc_init(xp, W0, b0.reshape(1, F))
    x0 = h
    cntp = _sc_cnt(ep)
    for i in range(NLAYER):
        beta = math.log(0.5 / (i + 1) + 1.0)
        xep = _sc_phase1(h, vp, ep)
        xe = _tc_combine(xep, cntp, degEp)
        xvp = _sc_phase2(xe, vp, ep)
        h = _tc_update(beta)(xvp, x0, degVp, Wc[i])
    return _tc_final(h[:N], Wout, bout.reshape(1, NCLASS))


# speedup vs baseline: 3.7139x; 1.0825x over previous
import functools
import math

import jax
import jax.numpy as jnp
from jax import lax
from jax.experimental import pallas as pl
from jax.experimental.pallas import tpu as pltpu
from jax.experimental.pallas import tpu_sc as plsc

N = 10000
M = 5000
NNZ = 320000
F = 128
NCLASS = 40
NLAYER = 2

NC = 2
NS = 16
NW = NC * NS
BATCH = 128
NB = -(-NNZ // (NW * BATCH))
NNZ_PAD = NW * NB * BATCH
PAD = NNZ_PAD - NNZ

N_PAD = 10240
M_PAD = 5120
ME16 = M_PAD // NS
NV16 = N_PAD // NS

_mesh = plsc.VectorSubcoreMesh(core_axis_name="c", subcore_axis_name="s")


def _fill(ref, width, value):
    def row(r, carry):
        for k in range(width // 16):
            ref[r, pl.ds(k * 16, 16)] = jnp.full((16,), value, jnp.float32)
        return carry
    lax.fori_loop(0, BATCH, row, 0)


def _chunks(total):
    out, off = [], 0
    while off < total:
        sz = min(BATCH, total - off)
        out.append((off, sz))
        off += sz
    return out


DEPTH = 4


def _pipelined_gather_scatter(src_hbm, gix, six, acc_sh, rows, gsem, ssem):
    for r in range(DEPTH):
        pltpu.make_async_copy(src_hbm.at[gix.at[r]], rows[r], gsem[r]).start()

    def outer(g, carry):
        for r in range(DEPTH):
            b = g * DEPTH + r

            @pl.when(b < NB)
            def _():
                pltpu.make_async_copy(
                    src_hbm.at[gix.at[b]], rows[r], gsem[r]).wait()
                sc = pltpu.make_async_copy(
                    rows[r], acc_sh.at[six.at[b]], ssem[r])
                sc.start(add=True)
                sc.wait()

                @pl.when(b + DEPTH < NB)
                def _():
                    pltpu.make_async_copy(
                        src_hbm.at[gix.at[b + DEPTH]], rows[r], gsem[r]
                    ).start()
        return carry

    lax.fori_loop(0, -(-NB // DEPTH), outer, 0)


@functools.partial(
    pl.kernel,
    out_type=jax.ShapeDtypeStruct((NC, M_PAD, F), jnp.float32),
    mesh=_mesh,
    scratch_types=[
        pltpu.VMEM((NB, BATCH), jnp.int32),
        pltpu.VMEM((NB, BATCH), jnp.int32),
    ] + [pltpu.VMEM((BATCH, F), jnp.float32)] * DEPTH
      + [pltpu.SemaphoreType.DMA] * (2 * DEPTH)
      + [pltpu.VMEM_SHARED((M_PAD, F), jnp.float32)],
)
def _sc_phase1(h_hbm, vix_hbm, eix_hbm, xep_out, vix, eix, *rest):
    rows, rest = rest[:DEPTH], rest[DEPTH:]
    gsem, rest = rest[:DEPTH], rest[DEPTH:]
    ssem, (xe_sh,) = rest[:DEPTH], rest[DEPTH:]
    c = lax.axis_index("c")
    s = lax.axis_index("s")
    w = c * NS + s
    _fill(rows[0], F, 0.0)
    for off, sz in _chunks(ME16):
        pltpu.sync_copy(rows[0].at[pl.ds(0, sz), :],
                        xe_sh.at[pl.ds(s * ME16 + off, sz), :])
    pltpu.sync_copy(vix_hbm.at[w], vix)
    pltpu.sync_copy(eix_hbm.at[w], eix)
    plsc.subcore_barrier()
    _pipelined_gather_scatter(h_hbm, vix, eix, xe_sh, rows, gsem, ssem)
    plsc.subcore_barrier()
    for off, sz in _chunks(ME16):
        pltpu.sync_copy(xe_sh.at[pl.ds(s * ME16 + off, sz), :],
                        rows[0].at[pl.ds(0, sz), :])
        pltpu.sync_copy(rows[0].at[pl.ds(0, sz), :],
                        xep_out.at[c, pl.ds(s * ME16 + off, sz), :])


@functools.partial(
    pl.kernel,
    out_type=jax.ShapeDtypeStruct((NC, M_PAD, F), jnp.float32),
    mesh=_mesh,
    scratch_types=[
        pltpu.VMEM((NB, BATCH), jnp.int32),
        pltpu.VMEM((BATCH, F), jnp.float32),
        pltpu.VMEM_SHARED((M_PAD, F), jnp.float32),
    ],
)
def _sc_cnt(eix_hbm, cnt_out, eix, ones_v, cnt_sh):
    c = lax.axis_index("c")
    s = lax.axis_index("s")
    w = c * NS + s
    _fill(ones_v, F, 0.0)
    for off, sz in _chunks(ME16):
        pltpu.sync_copy(ones_v.at[pl.ds(0, sz), :],
                        cnt_sh.at[pl.ds(s * ME16 + off, sz), :])
    _fill(ones_v, F, 1.0)
    pltpu.sync_copy(eix_hbm.at[w], eix)
    plsc.subcore_barrier()

    def body(b, carry):
        pltpu.sync_copy(ones_v, cnt_sh.at[eix.at[b]], add=True)
        return carry

    lax.fori_loop(0, NB, body, 0)
    plsc.subcore_barrier()
    for off, sz in _chunks(ME16):
        pltpu.sync_copy(cnt_sh.at[pl.ds(s * ME16 + off, sz), :],
                        ones_v.at[pl.ds(0, sz), :])
        pltpu.sync_copy(ones_v.at[pl.ds(0, sz), :],
                        cnt_out.at[c, pl.ds(s * ME16 + off, sz), :])


@functools.partial(
    pl.kernel,
    out_type=jax.ShapeDtypeStruct((NC, N_PAD, F), jnp.float32),
    mesh=_mesh,
    scratch_types=[
        pltpu.VMEM((NB, BATCH), jnp.int32),
        pltpu.VMEM((NB, BATCH), jnp.int32),
        pltpu.VMEM((BATCH, F), jnp.float32),
        pltpu.SemaphoreType.DMA,
        pltpu.VMEM_SHARED((N_PAD, F), jnp.float32),
    ],
)
def _sc_phase2(xe_hbm, vix_hbm, eix_hbm,
               xvp_out, vix, eix, rows, sem, xv_sh):
    c = lax.axis_index("c")
    s = lax.axis_index("s")
    w = c * NS + s
    _fill(rows, F, 0.0)
    for off, sz in _chunks(NV16):
        pltpu.sync_copy(rows.at[pl.ds(0, sz), :],
                        xv_sh.at[pl.ds(s * NV16 + off, sz), :])
    pltpu.sync_copy(vix_hbm.at[w], vix)
    pltpu.sync_copy(eix_hbm.at[w], eix)
    plsc.subcore_barrier()

    def body(b, carry):
        pltpu.async_copy(xe_hbm.at[eix.at[b]], rows, sem).wait()
        pltpu.sync_copy(rows, xv_sh.at[vix.at[b]], add=True)
        return carry

    lax.fori_loop(0, NB, body, 0)
    plsc.subcore_barrier()
    for off, sz in _chunks(NV16):
        pltpu.sync_copy(xv_sh.at[pl.ds(s * NV16 + off, sz), :],
                        rows)
        pltpu.sync_copy(rows,
                        xvp_out.at[c, pl.ds(s * NV16 + off, sz), :])


RB = 512


def _init_body(x_ref, w_ref, b_ref, o_ref):
    o_ref[...] = jax.nn.relu(
        jnp.dot(x_ref[...], w_ref[...], preferred_element_type=jnp.float32)
        + b_ref[...])


_tc_init = pl.pallas_call(
    _init_body,
    grid=(N_PAD // RB,),
    in_specs=[pl.BlockSpec((RB, F), lambda i: (i, 0)),
              pl.BlockSpec((F, F), lambda i: (0, 0)),
              pl.BlockSpec((1, F), lambda i: (0, 0))],
    out_specs=pl.BlockSpec((RB, F), lambda i: (i, 0)),
    out_shape=jax.ShapeDtypeStruct((N_PAD, F), jnp.float32),
)


def _combine_body(p_ref, c_ref, de_ref, o_ref):
    p = p_ref[0] + p_ref[1]
    cnt = c_ref[0, :, 0:1] + c_ref[1, :, 0:1]
    o_ref[...] = p * (de_ref[...] / jnp.maximum(cnt, 1.0))


_tc_combine = pl.pallas_call(
    _combine_body,
    grid=(M_PAD // RB,),
    in_specs=[pl.BlockSpec((NC, RB, F), lambda i: (0, i, 0)),
              pl.BlockSpec((NC, RB, F), lambda i: (0, i, 0)),
              pl.BlockSpec((RB, 1), lambda i: (i, 0))],
    out_specs=pl.BlockSpec((RB, F), lambda i: (i, 0)),
    out_shape=jax.ShapeDtypeStruct((M_PAD, F), jnp.float32),
)


def _update_body(p_ref, x0_ref, dv_ref, w_ref, o_ref, *, beta):
    xv = (p_ref[0] + p_ref[1]) * dv_ref[...]
    xi = 0.9 * xv + 0.1 * x0_ref[...]
    o_ref[...] = jax.nn.relu(
        (1.0 - beta) * xi
        + beta * jnp.dot(xi, w_ref[...], preferred_element_type=jnp.float32))


def _tc_update(beta):
    return pl.pallas_call(
        functools.partial(_update_body, beta=beta),
        grid=(N_PAD // RB,),
        in_specs=[pl.BlockSpec((NC, RB, F), lambda i: (0, i, 0)),
                  pl.BlockSpec((RB, F), lambda i: (i, 0)),
                  pl.BlockSpec((RB, 1), lambda i: (i, 0)),
                  pl.BlockSpec((F, F), lambda i: (0, 0))],
        out_specs=pl.BlockSpec((RB, F), lambda i: (i, 0)),
        out_shape=jax.ShapeDtypeStruct((N_PAD, F), jnp.float32),
    )


RO = 400


def _final_body(h_ref, w_ref, b_ref, o_ref):
    z = (jnp.dot(h_ref[...], w_ref[...], preferred_element_type=jnp.float32)
         + b_ref[...])
    m = jnp.max(z, axis=1, keepdims=True)
    lse = jnp.log(jnp.sum(jnp.exp(z - m), axis=1, keepdims=True))
    o_ref[...] = z - m - lse


_tc_final = pl.pallas_call(
    _final_body,
    grid=(N // RO,),
    in_specs=[pl.BlockSpec((RO, F), lambda i: (i, 0)),
              pl.BlockSpec((F, NCLASS), lambda i: (0, 0)),
              pl.BlockSpec((1, NCLASS), lambda i: (0, 0))],
    out_specs=pl.BlockSpec((RO, NCLASS), lambda i: (i, 0)),
    out_shape=jax.ShapeDtypeStruct((N, NCLASS), jnp.float32),
)


def kernel(x, V, E, degV, degE, W0, b0, Wc, Wout, bout):
    V32 = V.astype(jnp.int32)
    E32 = E.astype(jnp.int32)
    vp = jnp.concatenate([V32, jnp.full((PAD,), N, jnp.int32)]).reshape(NW, NB, BATCH)
    ep = jnp.concatenate([E32, jnp.full((PAD,), M, jnp.int32)]).reshape(NW, NB, BATCH)
    xp = jnp.pad(x, ((0, N_PAD - N), (0, 0)))
    degVp = jnp.pad(degV.astype(jnp.float32), ((0, N_PAD - N), (0, 0)))
    degEp = jnp.pad(degE.astype(jnp.float32), ((0, M_PAD - M), (0, 0)))
    h = _tc_init(xp, W0, b0.reshape(1, F))
    x0 = h
    cntp = _sc_cnt(ep)
    for i in range(NLAYER):
        beta = math.log(0.5 / (i + 1) + 1.0)
        xep = _sc_phase1(h, vp, ep)
        xe = _tc_combine(xep, cntp, degEp)
        xvp = _sc_phase2(xe, vp, ep)
        h = _tc_update(beta)(xvp, x0, degVp, Wc[i])
    return _tc_final(h[:N], Wout, bout.reshape(1, NCLASS))
